# Initial kernel scaffold; baseline (speedup 1.0000x reference)
import jax
import jax.numpy as jnp
from jax.experimental import pallas as pl


def kernel(feat_ge, feat_a_ge, feat_deconv, feat_a_deconv, edge_index, W_enc_ge, W_dec_ge, W_enc_dc, W_dec_dc, W_b, W_gate, b_gate, W_dec_lat, W_g1, a_src1, a_dst1, W_g2, a_src2, a_dst2):
    raise NotImplementedError("write your pallas kernel here")


# speedup vs baseline: 13.0042x; 13.0042x over previous
import jax
import jax.numpy as jnp
from jax import lax
from jax.experimental import pallas as pl
from jax.experimental.pallas import tpu as pltpu
from jax.experimental.pallas import tpu_sc as plsc

N = 10000
E = 320000
CHUNK = 128
TILES = 16
CPT = 160
EP = CPT * CHUNK * TILES
ROWS_E = EP // CHUNK
N_ACC = TILES * 632
ZROWS = 632
OPT = 624
W = 64
B = 1000
GRID = N // B

_mesh = plsc.VectorSubcoreMesh(core_axis_name="c", subcore_axis_name="s")
_f32 = jnp.float32


def _sds(*shape):
    return jax.ShapeDtypeStruct(shape, _f32)


def _fill(ref, rows, width, value):
    v = jnp.full((16,), value, _f32)
    for r in range(rows):
        for k in range(width // 16):
            ref[r, pl.ds(k * 16, 16)] = v


def _zero_acc(zb, acc, s):
    nz = zb.shape[0]
    base = s * ZROWS
    for i in range((ZROWS + nz - 1) // nz):
        off = min(i * nz, ZROWS - nz)
        pltpu.sync_copy(zb, acc.at[pl.ds(base + off, nz)])


def _load_idx(srcp, dstp, src_v, dst_v, s):
    pltpu.sync_copy(srcp.at[pl.ds(s * CPT, CPT)], src_v)
    pltpu.sync_copy(dstp.at[pl.ds(s * CPT, CPT)], dst_v)


def _copy_rows(src, dst, s):
    row0 = s * OPT
    pltpu.sync_copy(src.at[pl.ds(row0, OPT)], dst.at[pl.ds(row0, OPT)])
    tail = N - TILES * OPT

    @pl.when(s == TILES - 1)
    def _():
        pltpu.sync_copy(src.at[pl.ds(TILES * OPT, tail)],
                        dst.at[pl.ds(TILES * OPT, tail)])


def _make_spmm(n_lo, n_hi, with_deg):
    n_tot = n_lo + n_hi
    outs = [_sds(N, W) for _ in range(n_tot)] + ([_sds(N, 16)] if with_deg else [])
    scratch = [
        pltpu.VMEM((CPT, CHUNK), jnp.int32),
        pltpu.VMEM((CPT, CHUNK), jnp.int32),
        pltpu.VMEM((CHUNK, W), _f32),
        pltpu.VMEM((16, W), _f32),
        pltpu.VMEM((CHUNK, 16), _f32),
        pltpu.VMEM((16, 16), _f32),
        pltpu.VMEM_SHARED((N_ACC, W), _f32),
        pltpu.VMEM_SHARED((N_ACC, 16), _f32),
        pltpu.SemaphoreType.DMA,
    ]

    def body(*refs):
        xs = refs[:n_tot]
        srcp, dstp = refs[n_tot], refs[n_tot + 1]
        o = n_tot + 2
        outs_r = refs[o:o + n_tot]
        o += n_tot
        deg_out = refs[o] if with_deg else None
        o += 1 if with_deg else 0
        src_v, dst_v, rows_v, zb, ones_v, zbd, acc, dacc, sem = refs[o:]
        c = lax.axis_index("c")
        s = lax.axis_index("s")
        _load_idx(srcp, dstp, src_v, dst_v, s)
        _fill(zb, 16, W, 0.0)
        if with_deg:
            _fill(ones_v, CHUNK, 16, 1.0)
            _fill(zbd, 16, 16, 0.0)

        def run_group(x_ref, out_ref, do_deg):
            _zero_acc(zb, acc, s)
            if do_deg:
                nz = 16
                base = s * ZROWS
                for i in range((ZROWS + nz - 1) // nz):
                    off = min(i * nz, ZROWS - nz)
                    pltpu.sync_copy(zbd, dacc.at[pl.ds(base + off, nz)])
            plsc.subcore_barrier()

            def step(j, carry):
                pltpu.async_copy(x_ref.at[src_v.at[j]], rows_v, sem).wait()
                pltpu.sync_copy(rows_v, acc.at[dst_v.at[j]], add=True)
                if do_deg:
                    pltpu.sync_copy(ones_v, dacc.at[dst_v.at[j]], add=True)
                return carry
            lax.fori_loop(0, CPT, step, 0)
            plsc.subcore_barrier()
            _copy_rows(acc, out_ref, s)
            if do_deg:
                _copy_rows(dacc, deg_out, s)

        @pl.when(c == 0)
        def _():
            for g in range(n_lo):
                run_group(xs[g], outs_r[g], with_deg and g == 0)

        @pl.when(c == 1)
        def _():
            for g in range(n_hi):
                run_group(xs[n_lo + g], outs_r[n_lo + g], False)

    return pl.kernel(body, out_type=tuple(outs), mesh=_mesh, scratch_types=scratch,
                     compiler_params=pltpu.CompilerParams(use_tc_tiling_on_sc=False))


def _make_gat(hw, n_heads, heads_lo, heads_hi):
    outs = [_sds(N, hw) for _ in range(n_heads)] + [_sds(2, N, 16)]
    scratch = [
        pltpu.VMEM((CPT, CHUNK), jnp.int32),
        pltpu.VMEM((CPT, CHUNK), jnp.int32),
        pltpu.VMEM((CHUNK, 16), _f32),
        pltpu.VMEM((CHUNK, 16), _f32),
        pltpu.VMEM((CHUNK, 16), _f32),
        pltpu.VMEM((CHUNK, hw), _f32),
        pltpu.VMEM((16, hw), _f32),
        pltpu.VMEM((16, 16), _f32),
        pltpu.VMEM_SHARED((N_ACC, hw), _f32),
        pltpu.VMEM_SHARED((N_ACC, 16), _f32),
        pltpu.SemaphoreType.DMA,
    ]

    def body(*refs):
        xs = refs[:n_heads]
        als, ald, srcp, dstp = refs[n_heads:n_heads + 4]
        o = n_heads + 4
        outs_r = refs[o:o + n_heads]
        den_out = refs[o + n_heads]
        (src_v, dst_v, als_v, ald_v, s_v, h_v, zb, zbd, acc, dacc,
         sem) = refs[o + n_heads + 1:]
        c = lax.axis_index("c")
        s = lax.axis_index("s")
        _load_idx(srcp, dstp, src_v, dst_v, s)
        _fill(zb, 16, hw, 0.0)
        _fill(zbd, 16, 16, 0.0)

        def run_head(x_ref, out_ref, lane, do_den, den_slot):
            _zero_acc(zb, acc, s)
            if do_den:
                nz = 16
                base = s * ZROWS
                for i in range((ZROWS + nz - 1) // nz):
                    off = min(i * nz, ZROWS - nz)
                    pltpu.sync_copy(zbd, dacc.at[pl.ds(base + off, nz)])
            plsc.subcore_barrier()

            def step(j, carry):
                pltpu.async_copy(als.at[src_v.at[j]], als_v, sem).wait()
                pltpu.async_copy(ald.at[dst_v.at[j]], ald_v, sem).wait()
                pltpu.async_copy(x_ref.at[src_v.at[j]], h_v, sem).wait()

                def erow(e, cc):
                    a = als_v[e] + ald_v[e]
                    a = jnp.where(a > 0.0, a, 0.2 * a)
                    s_v[e] = jnp.exp(a)
                    return cc
                lax.fori_loop(0, CHUNK, erow, 0)
                if do_den:
                    pltpu.sync_copy(s_v, dacc.at[dst_v.at[j]], add=True)

                def mrow(e, cc):
                    sv = s_v[e]
                    se = sv[lane]
                    for k in range(hw // 16):
                        h_v[e, pl.ds(k * 16, 16)] = h_v[e, pl.ds(k * 16, 16)] * se
                    return cc
                lax.fori_loop(0, CHUNK, mrow, 0)
                pltpu.sync_copy(h_v, acc.at[dst_v.at[j]], add=True)
                return carry
            lax.fori_loop(0, CPT, step, 0)
            plsc.subcore_barrier()
            _copy_rows(acc, out_ref, s)
            if do_den:
                _copy_rows(dacc, den_out.at[den_slot], s)

        @pl.when(c == 0)
        def _():
            for i, (xi, lane) in enumerate(heads_lo):
                run_head(xs[xi], outs_r[xi], lane, i == 0, 0)

        @pl.when(c == 1)
        def _():
            for i, (xi, lane) in enumerate(heads_hi):
                run_head(xs[xi], outs_r[xi], lane, i == 0, 1)

    return pl.kernel(body, out_type=tuple(outs), mesh=_mesh, scratch_types=scratch,
                     compiler_params=pltpu.CompilerParams(use_tc_tiling_on_sc=False))


def _row_spec(w):
    return pl.BlockSpec((B, w), lambda i: (i, 0))


def _full_spec(a, b):
    return pl.BlockSpec((a, b), lambda i: (0, 0))


def _tc1_body(fg, fa, fd, fad, Wge, Wdc, m0, m1, m2, m3):
    m0[...] = fg[...] @ Wge[...]
    m1[...] = fa[...] @ Wge[...]
    m2[...] = fd[...] @ Wdc[...]
    m3[...] = fad[...] @ Wdc[...]


def _tc1(fg, fa, fd, fad, Wge, Wdc):
    return pl.pallas_call(
        _tc1_body,
        grid=(GRID,),
        in_specs=[_row_spec(128), _row_spec(128), _row_spec(64), _row_spec(64),
                  _full_spec(128, 64), _full_spec(64, 64)],
        out_specs=[_row_spec(64)] * 4,
        out_shape=[_sds(N, 64)] * 4,
    )(fg, fa, fd, fad, Wge, Wdc)


def _l2n(x):
    return x / jnp.clip(jnp.sqrt(jnp.sum(x * x, axis=-1, keepdims=True)), 1e-12)


def _tc2_body(Sge, Sage, Sdc, Sadc, dega, Wg, bg, Wg1, as1, ad1,
              e_ge_o, e_age_o, e_dc_o, e_adc_o, ge_n_o, dc_n_o, lat_o,
              h0, h1, h2, h3, als_o, ald_o):
    deg = jnp.maximum(dega[:, 0:1], 1.0)
    e_ge = jax.nn.relu(Sge[...] / deg)
    e_age = jax.nn.relu(Sage[...] / deg)
    e_dc = jax.nn.relu(Sdc[...] / deg)
    e_adc = jax.nn.relu(Sadc[...] / deg)
    e_ge_o[...] = e_ge
    e_age_o[...] = e_age
    e_dc_o[...] = e_dc
    e_adc_o[...] = e_adc
    gn = _l2n(e_ge)
    dn = _l2n(e_dc)
    ge_n_o[...] = gn
    dc_n_o[...] = dn
    gate = jax.nn.sigmoid(jnp.concatenate([gn, dn], axis=1) @ Wg[...] + bg[...])
    el = gate * gn + (1.0 - gate) * dn
    lat_o[...] = el
    h = el @ Wg1[...]
    h0[...] = h[:, 0:64]
    h1[...] = h[:, 64:128]
    h2[...] = h[:, 128:192]
    h3[...] = h[:, 192:256]
    zs = jnp.zeros((B, 12), _f32)
    als_o[...] = jnp.concatenate(
        [h[:, 64 * k:64 * k + 64] @ as1[...][k][:, None] for k in range(4)] + [zs],
        axis=1)
    ald_o[...] = jnp.concatenate(
        [h[:, 64 * k:64 * k + 64] @ ad1[...][k][:, None] for k in range(4)] + [zs],
        axis=1)


def _tc2(Sge, Sage, Sdc, Sadc, dega, Wg, bg, Wg1, as1, ad1):
    return pl.pallas_call(
        _tc2_body,
        grid=(GRID,),
        in_specs=[_row_spec(64)] * 4 + [_row_spec(16),
                  _full_spec(128, 64), _full_spec(1, 64), _full_spec(64, 256),
                  _full_spec(4, 64), _full_spec(4, 64)],
        out_specs=[_row_spec(64)] * 11 + [_row_spec(16), _row_spec(16)],
        out_shape=[_sds(N, 64)] * 11 + [_sds(N, 16), _sds(N, 16)],
    )(Sge, Sage, Sdc, Sadc, dega, Wg, bg, Wg1, as1, ad1)


def _tc3_body(S2ge, S2age, S2dc, S2adc, S2lat, dega, e_ge, e_age, e_dc, e_adc,
              g0, g1, g2, g3, den1, Wdge, Wddc, Wdlat, Wb, Wg2, as2, ad2,
              h_ge_o, h_dcv_o, h_lat_o, rets, h2a, h2b, als2, ald2):
    deg = jnp.maximum(dega[:, 0:1], 1.0)
    Pge = S2ge[...] / deg
    Page = S2age[...] / deg
    Pdc = S2dc[...] / deg
    Padc = S2adc[...] / deg
    Plat = S2lat[...] / deg
    h_ge_o[...] = Pge @ Wdge[...]
    h_dcv_o[...] = Pdc @ Wddc[...]
    h_lat_o[...] = Plat @ Wdlat[...]

    def disc(g, hp, hm):
        cw = g @ Wb[...]
        return (jnp.sum(hp * cw, axis=-1, keepdims=True),
                jnp.sum(hm * cw, axis=-1, keepdims=True))

    a1, a2 = disc(jax.nn.sigmoid(Pge), e_ge[...], e_age[...])
    b1, b2 = disc(jax.nn.sigmoid(Page), e_age[...], e_ge[...])
    c1, c2 = disc(jax.nn.sigmoid(Pdc), e_dc[...], e_adc[...])
    d1, d2 = disc(jax.nn.sigmoid(Padc), e_adc[...], e_dc[...])
    rets[...] = jnp.concatenate([a1, a2, b1, b2, c1, c2, d1, d2], axis=1)
    den = den1[...] + 1e-16
    h1v = jnp.concatenate(
        [g[...] / den[:, k:k + 1] for k, g in enumerate((g0, g1, g2, g3))], axis=1)
    h1v = jnp.where(h1v > 0.0, h1v, jnp.exp(jnp.minimum(h1v, 0.0)) - 1.0)
    h2 = h1v @ Wg2[...]
    h2a[...] = h2[:, 0:16]
    h2b[...] = h2[:, 16:32]
    zs = jnp.zeros((B, 15), _f32)
    als2[...] = jnp.concatenate([h2 @ as2[...][0][:, None], zs], axis=1)
    ald2[...] = jnp.concatenate([h2 @ ad2[...][0][:, None], zs], axis=1)


def _tc3(S2ge, S2age, S2dc, S2adc, S2lat, dega, e_ge, e_age, e_dc, e_adc,
         g0, g1, g2, g3, den1, Wdge, Wddc, Wdlat, Wb, Wg2, as2, ad2):
    return pl.pallas_call(
        _tc3_body,
        grid=(GRID,),
        in_specs=[_row_spec(64)] * 5 + [_row_spec(16)] + [_row_spec(64)] * 8
                 + [_row_spec(16),
                    _full_spec(64, 128), _full_spec(64, 64), _full_spec(64, 128),
                    _full_spec(64, 64), _full_spec(256, 32),
                    _full_spec(1, 32), _full_spec(1, 32)],
        out_specs=[_row_spec(128), _row_spec(64), _row_spec(128), _row_spec(8),
                   _row_spec(16), _row_spec(16), _row_spec(16), _row_spec(16)],
        out_shape=[_sds(N, 128), _sds(N, 64), _sds(N, 128), _sds(N, 8),
                   _sds(N, 16), _sds(N, 16), _sds(N, 16), _sds(N, 16)],
    )(S2ge, S2age, S2dc, S2adc, S2lat, dega, e_ge, e_age, e_dc, e_adc,
      g0, g1, g2, g3, den1, Wdge, Wddc, Wdlat, Wb, Wg2, as2, ad2)


def _tc4_body(g2a, g2b, den2, x_out):
    den = den2[:, 0:1] + 1e-16
    x_out[...] = jnp.concatenate([g2a[...], g2b[...]], axis=1) / den


def _tc4(g2a, g2b, den2):
    return pl.pallas_call(
        _tc4_body,
        grid=(GRID,),
        in_specs=[_row_spec(16), _row_spec(16), _row_spec(16)],
        out_specs=_row_spec(32),
        out_shape=_sds(N, 32),
    )(g2a, g2b, den2)


_spmm_p1 = _make_spmm(2, 2, True)
_spmm_p2 = _make_spmm(3, 2, False)
_gat1 = _make_gat(64, 4, ((0, 0), (1, 1)), ((2, 2), (3, 3)))
_gat2 = _make_gat(16, 2, ((0, 0),), ((1, 0),))


def kernel(feat_ge, feat_a_ge, feat_deconv, feat_a_deconv, edge_index,
           W_enc_ge, W_dec_ge, W_enc_dc, W_dec_dc, W_b, W_gate, b_gate,
           W_dec_lat, W_g1, a_src1, a_dst1, W_g2, a_src2, a_dst2):
    src = edge_index[0]
    dst = edge_index[1]
    pad = EP - E
    srcp = jnp.concatenate([src, jnp.zeros((pad,), jnp.int32)]).reshape(ROWS_E, CHUNK)
    dstp = jnp.concatenate([dst, jnp.full((pad,), N, jnp.int32)]).reshape(ROWS_E, CHUNK)

    m_ge, m_age, m_dc, m_adc = _tc1(feat_ge, feat_a_ge, feat_deconv,
                                    feat_a_deconv, W_enc_ge, W_enc_dc)
    Sge, Sage, Sdc, Sadc, dega = _spmm_p1(m_ge, m_age, m_dc, m_adc, srcp, dstp)
    (e_ge, e_age, e_dc, e_adc, ge_n, dc_n, emb_lat, h0, h1, h2, h3,
     als, ald) = _tc2(Sge, Sage, Sdc, Sadc, dega, W_gate,
                      b_gate.reshape(1, 64), W_g1, a_src1, a_dst1)
    S2ge, S2age, S2dc, S2adc, S2lat = _spmm_p2(e_ge, e_age, e_dc, e_adc,
                                               emb_lat, srcp, dstp)
    g0, g1, g2, g3, den1 = _gat1(h0, h1, h2, h3, als, ald, srcp, dstp)
    (h_ge, h_dcv, h_lat, rets, h2a, h2b, als2, ald2) = _tc3(
        S2ge, S2age, S2dc, S2adc, S2lat, dega, e_ge, e_age, e_dc, e_adc,
        g0, g1, g2, g3, den1[0], W_dec_ge, W_dec_dc, W_dec_lat, W_b, W_g2,
        a_src2, a_dst2)
    g2a, g2b, den2 = _gat2(h2a, h2b, als2, ald2, srcp, dstp)
    x_out = _tc4(g2a, g2b, den2[0])

    return (ge_n, h_ge, rets[:, 0:2], rets[:, 2:4], dc_n, h_dcv,
            rets[:, 4:6], rets[:, 6:8], emb_lat, h_lat, x_out)


# speedup vs baseline: 20.8607x; 1.6041x over previous
import jax
import jax.numpy as jnp
from jax import lax
from jax.experimental import pallas as pl
from jax.experimental.pallas import tpu as pltpu
from jax.experimental.pallas import tpu_sc as plsc

N = 10000
E = 320000
CHUNK = 128
TILES = 16
CPT = 160
EP = CPT * CHUNK * TILES
ROWS_E = EP // CHUNK
N_ACC = TILES * 632
ZROWS = 632
OPT = 624
W = 64
B = 1000
GRID = N // B

_mesh = plsc.VectorSubcoreMesh(core_axis_name="c", subcore_axis_name="s")
_f32 = jnp.float32


def _sds(*shape):
    return jax.ShapeDtypeStruct(shape, _f32)


def _fill(ref, rows, width, value):
    v = jnp.full((16,), value, _f32)
    for r in range(rows):
        for k in range(width // 16):
            ref[r, pl.ds(k * 16, 16)] = v


def _zero_acc(zb, acc, s):
    nz = zb.shape[0]
    base = s * ZROWS
    for i in range((ZROWS + nz - 1) // nz):
        off = min(i * nz, ZROWS - nz)
        pltpu.sync_copy(zb, acc.at[pl.ds(base + off, nz)])


def _load_idx(srcp, dstp, src_v, dst_v, s):
    pltpu.sync_copy(srcp.at[pl.ds(s * CPT, CPT)], src_v)
    pltpu.sync_copy(dstp.at[pl.ds(s * CPT, CPT)], dst_v)


def _copy_rows(src, dst, s):
    row0 = s * OPT
    pltpu.sync_copy(src.at[pl.ds(row0, OPT)], dst.at[pl.ds(row0, OPT)])
    tail = N - TILES * OPT

    @pl.when(s == TILES - 1)
    def _():
        pltpu.sync_copy(src.at[pl.ds(TILES * OPT, tail)],
                        dst.at[pl.ds(TILES * OPT, tail)])


def _make_spmm(n_lo, n_hi, with_deg):
    n_tot = n_lo + n_hi
    outs = [_sds(N, W) for _ in range(n_tot)] + ([_sds(N, 16)] if with_deg else [])
    scratch = [
        pltpu.VMEM((CPT, CHUNK), jnp.int32),
        pltpu.VMEM((CPT, CHUNK), jnp.int32),
        pltpu.VMEM((CHUNK, W), _f32),
        pltpu.VMEM((CHUNK, W), _f32),
        pltpu.VMEM((16, W), _f32),
        pltpu.VMEM((CHUNK, 16), _f32),
        pltpu.VMEM((16, 16), _f32),
        pltpu.VMEM_SHARED((N_ACC, W), _f32),
        pltpu.VMEM_SHARED((N_ACC, 16), _f32),
        pltpu.SemaphoreType.DMA,
        pltpu.SemaphoreType.DMA,
    ]

    def body(*refs):
        xs = refs[:n_tot]
        srcp, dstp = refs[n_tot], refs[n_tot + 1]
        o = n_tot + 2
        outs_r = refs[o:o + n_tot]
        o += n_tot
        deg_out = refs[o] if with_deg else None
        o += 1 if with_deg else 0
        (src_v, dst_v, rows0, rows1, zb, ones_v, zbd, acc, dacc,
         sem0, sem1) = refs[o:]
        c = lax.axis_index("c")
        s = lax.axis_index("s")
        _load_idx(srcp, dstp, src_v, dst_v, s)
        _fill(zb, 16, W, 0.0)
        if with_deg:
            _fill(ones_v, CHUNK, 16, 1.0)
            _fill(zbd, 16, 16, 0.0)

        def run_group(x_ref, out_ref, do_deg):
            _zero_acc(zb, acc, s)
            if do_deg:
                nz = 16
                base = s * ZROWS
                for i in range((ZROWS + nz - 1) // nz):
                    off = min(i * nz, ZROWS - nz)
                    pltpu.sync_copy(zbd, dacc.at[pl.ds(base + off, nz)])
            plsc.subcore_barrier()

            def issue(j, buf, sem):
                pltpu.async_copy(x_ref.at[src_v.at[j]], buf, sem)

            def wait(buf, sem):
                pltpu.make_async_copy(x_ref.at[src_v.at[0]], buf, sem).wait()

            def drain(j, buf, sem):
                wait(buf, sem)
                pltpu.sync_copy(buf, acc.at[dst_v.at[j]], add=True)
                if do_deg:
                    pltpu.sync_copy(ones_v, dacc.at[dst_v.at[j]], add=True)

            issue(0, rows0, sem0)
            issue(1, rows1, sem1)

            def step(jj, carry):
                j0 = 2 * jj
                drain(j0, rows0, sem0)

                @pl.when(j0 + 2 < CPT)
                def _():
                    issue(j0 + 2, rows0, sem0)

                drain(j0 + 1, rows1, sem1)

                @pl.when(j0 + 3 < CPT)
                def _():
                    issue(j0 + 3, rows1, sem1)
                return carry
            lax.fori_loop(0, CPT // 2, step, 0)
            plsc.subcore_barrier()
            _copy_rows(acc, out_ref, s)
            if do_deg:
                _copy_rows(dacc, deg_out, s)

        @pl.when(c == 0)
        def _():
            for g in range(n_lo):
                run_group(xs[g], outs_r[g], with_deg and g == 0)

        @pl.when(c == 1)
        def _():
            for g in range(n_hi):
                run_group(xs[n_lo + g], outs_r[n_lo + g], False)

    return pl.kernel(body, out_type=tuple(outs), mesh=_mesh, scratch_types=scratch,
                     compiler_params=pltpu.CompilerParams(use_tc_tiling_on_sc=False))


def _make_gat(hw, n_heads, heads_lo, heads_hi):
    outs = [_sds(N, hw) for _ in range(n_heads)] + [_sds(2, N, 16)]
    scratch = [
        pltpu.VMEM((CPT, CHUNK), jnp.int32),
        pltpu.VMEM((CPT, CHUNK), jnp.int32),
        pltpu.VMEM((CHUNK, 16), _f32),
        pltpu.VMEM((CHUNK, 16), _f32),
        pltpu.VMEM((CHUNK, 16), _f32),
        pltpu.VMEM((CHUNK, 16), _f32),
        pltpu.VMEM((CHUNK, 16), _f32),
        pltpu.VMEM((CHUNK, hw), _f32),
        pltpu.VMEM((CHUNK, hw), _f32),
        pltpu.VMEM((16, hw), _f32),
        pltpu.VMEM((16, 16), _f32),
        pltpu.VMEM_SHARED((N_ACC, hw), _f32),
        pltpu.VMEM_SHARED((N_ACC, 16), _f32),
        pltpu.SemaphoreType.DMA,
        pltpu.SemaphoreType.DMA,
    ]

    def body(*refs):
        xs = refs[:n_heads]
        als, ald, srcp, dstp = refs[n_heads:n_heads + 4]
        o = n_heads + 4
        outs_r = refs[o:o + n_heads]
        den_out = refs[o + n_heads]
        (src_v, dst_v, als0, als1, ald0, ald1, s_v, h0, h1, zb, zbd, acc, dacc,
         sem0, sem1) = refs[o + n_heads + 1:]
        c = lax.axis_index("c")
        s = lax.axis_index("s")
        _load_idx(srcp, dstp, src_v, dst_v, s)
        _fill(zb, 16, hw, 0.0)
        _fill(zbd, 16, 16, 0.0)

        def run_head(x_ref, out_ref, lane, do_den, den_slot):
            _zero_acc(zb, acc, s)
            if do_den:
                nz = 16
                base = s * ZROWS
                for i in range((ZROWS + nz - 1) // nz):
                    off = min(i * nz, ZROWS - nz)
                    pltpu.sync_copy(zbd, dacc.at[pl.ds(base + off, nz)])
            plsc.subcore_barrier()

            def issue(j, als_b, ald_b, h_b, sem):
                pltpu.async_copy(als.at[src_v.at[j]], als_b, sem)
                pltpu.async_copy(ald.at[dst_v.at[j]], ald_b, sem)
                pltpu.async_copy(x_ref.at[src_v.at[j]], h_b, sem)

            def wait(als_b, ald_b, h_b, sem):
                pltpu.make_async_copy(als.at[src_v.at[0]], als_b, sem).wait()
                pltpu.make_async_copy(ald.at[dst_v.at[0]], ald_b, sem).wait()
                pltpu.make_async_copy(x_ref.at[src_v.at[0]], h_b, sem).wait()

            def drain(j, als_b, ald_b, h_b, sem):
                wait(als_b, ald_b, h_b, sem)

                def erows(q, cc):
                    for u in range(4):
                        e = 4 * q + u
                        a = als_b[e] + ald_b[e]
                        a = jnp.where(a > 0.0, a, 0.2 * a)
                        sv = jnp.exp(a)
                        if do_den:
                            s_v[e] = sv
                        se = sv[lane]
                        for k in range(hw // 16):
                            h_b[e, pl.ds(k * 16, 16)] = (
                                h_b[e, pl.ds(k * 16, 16)] * se)
                    return cc
                lax.fori_loop(0, CHUNK // 4, erows, 0)
                if do_den:
                    pltpu.sync_copy(s_v, dacc.at[dst_v.at[j]], add=True)
                pltpu.sync_copy(h_b, acc.at[dst_v.at[j]], add=True)

            issue(0, als0, ald0, h0, sem0)
            issue(1, als1, ald1, h1, sem1)

            def step(jj, carry):
                j0 = 2 * jj
                drain(j0, als0, ald0, h0, sem0)

                @pl.when(j0 + 2 < CPT)
                def _():
                    issue(j0 + 2, als0, ald0, h0, sem0)

                drain(j0 + 1, als1, ald1, h1, sem1)

                @pl.when(j0 + 3 < CPT)
                def _():
                    issue(j0 + 3, als1, ald1, h1, sem1)
                return carry
            lax.fori_loop(0, CPT // 2, step, 0)
            plsc.subcore_barrier()
            _copy_rows(acc, out_ref, s)
            if do_den:
                _copy_rows(dacc, den_out.at[den_slot], s)

        @pl.when(c == 0)
        def _():
            for i, (xi, lane) in enumerate(heads_lo):
                run_head(xs[xi], outs_r[xi], lane, i == 0, 0)

        @pl.when(c == 1)
        def _():
            for i, (xi, lane) in enumerate(heads_hi):
                run_head(xs[xi], outs_r[xi], lane, i == 0, 1)

    return pl.kernel(body, out_type=tuple(outs), mesh=_mesh, scratch_types=scratch,
                     compiler_params=pltpu.CompilerParams(use_tc_tiling_on_sc=False))


def _row_spec(w):
    return pl.BlockSpec((B, w), lambda i: (i, 0))


def _full_spec(a, b):
    return pl.BlockSpec((a, b), lambda i: (0, 0))


def _tc1_body(fg, fa, fd, fad, Wge, Wdc, m0, m1, m2, m3):
    m0[...] = fg[...] @ Wge[...]
    m1[...] = fa[...] @ Wge[...]
    m2[...] = fd[...] @ Wdc[...]
    m3[...] = fad[...] @ Wdc[...]


def _tc1(fg, fa, fd, fad, Wge, Wdc):
    return pl.pallas_call(
        _tc1_body,
        grid=(GRID,),
        in_specs=[_row_spec(128), _row_spec(128), _row_spec(64), _row_spec(64),
                  _full_spec(128, 64), _full_spec(64, 64)],
        out_specs=[_row_spec(64)] * 4,
        out_shape=[_sds(N, 64)] * 4,
    )(fg, fa, fd, fad, Wge, Wdc)


def _l2n(x):
    return x / jnp.clip(jnp.sqrt(jnp.sum(x * x, axis=-1, keepdims=True)), 1e-12)


def _tc2_body(Sge, Sage, Sdc, Sadc, dega, Wg, bg, Wg1, as1, ad1,
              e_ge_o, e_age_o, e_dc_o, e_adc_o, ge_n_o, dc_n_o, lat_o,
              h0, h1, h2, h3, als_o, ald_o):
    deg = jnp.maximum(dega[:, 0:1], 1.0)
    e_ge = jax.nn.relu(Sge[...] / deg)
    e_age = jax.nn.relu(Sage[...] / deg)
    e_dc = jax.nn.relu(Sdc[...] / deg)
    e_adc = jax.nn.relu(Sadc[...] / deg)
    e_ge_o[...] = e_ge
    e_age_o[...] = e_age
    e_dc_o[...] = e_dc
    e_adc_o[...] = e_adc
    gn = _l2n(e_ge)
    dn = _l2n(e_dc)
    ge_n_o[...] = gn
    dc_n_o[...] = dn
    gate = jax.nn.sigmoid(jnp.concatenate([gn, dn], axis=1) @ Wg[...] + bg[...])
    el = gate * gn + (1.0 - gate) * dn
    lat_o[...] = el
    h = el @ Wg1[...]
    h0[...] = h[:, 0:64]
    h1[...] = h[:, 64:128]
    h2[...] = h[:, 128:192]
    h3[...] = h[:, 192:256]
    zs = jnp.zeros((B, 12), _f32)
    als_o[...] = jnp.concatenate(
        [h[:, 64 * k:64 * k + 64] @ as1[...][k][:, None] for k in range(4)] + [zs],
        axis=1)
    ald_o[...] = jnp.concatenate(
        [h[:, 64 * k:64 * k + 64] @ ad1[...][k][:, None] for k in range(4)] + [zs],
        axis=1)


def _tc2(Sge, Sage, Sdc, Sadc, dega, Wg, bg, Wg1, as1, ad1):
    return pl.pallas_call(
        _tc2_body,
        grid=(GRID,),
        in_specs=[_row_spec(64)] * 4 + [_row_spec(16),
                  _full_spec(128, 64), _full_spec(1, 64), _full_spec(64, 256),
                  _full_spec(4, 64), _full_spec(4, 64)],
        out_specs=[_row_spec(64)] * 11 + [_row_spec(16), _row_spec(16)],
        out_shape=[_sds(N, 64)] * 11 + [_sds(N, 16), _sds(N, 16)],
    )(Sge, Sage, Sdc, Sadc, dega, Wg, bg, Wg1, as1, ad1)


def _tc3_body(S2ge, S2age, S2dc, S2adc, S2lat, dega, e_ge, e_age, e_dc, e_adc,
              g0, g1, g2, g3, den1, Wdge, Wddc, Wdlat, Wb, Wg2, as2, ad2,
              h_ge_o, h_dcv_o, h_lat_o, rets, h2a, h2b, als2, ald2):
    deg = jnp.maximum(dega[:, 0:1], 1.0)
    Pge = S2ge[...] / deg
    Page = S2age[...] / deg
    Pdc = S2dc[...] / deg
    Padc = S2adc[...] / deg
    Plat = S2lat[...] / deg
    h_ge_o[...] = Pge @ Wdge[...]
    h_dcv_o[...] = Pdc @ Wddc[...]
    h_lat_o[...] = Plat @ Wdlat[...]

    def disc(g, hp, hm):
        cw = g @ Wb[...]
        return (jnp.sum(hp * cw, axis=-1, keepdims=True),
                jnp.sum(hm * cw, axis=-1, keepdims=True))

    a1, a2 = disc(jax.nn.sigmoid(Pge), e_ge[...], e_age[...])
    b1, b2 = disc(jax.nn.sigmoid(Page), e_age[...], e_ge[...])
    c1, c2 = disc(jax.nn.sigmoid(Pdc), e_dc[...], e_adc[...])
    d1, d2 = disc(jax.nn.sigmoid(Padc), e_adc[...], e_dc[...])
    rets[...] = jnp.concatenate([a1, a2, b1, b2, c1, c2, d1, d2], axis=1)
    den = den1[...] + 1e-16
    h1v = jnp.concatenate(
        [g[...] / den[:, k:k + 1] for k, g in enumerate((g0, g1, g2, g3))], axis=1)
    h1v = jnp.where(h1v > 0.0, h1v, jnp.exp(jnp.minimum(h1v, 0.0)) - 1.0)
    h2 = h1v @ Wg2[...]
    h2a[...] = h2[:, 0:16]
    h2b[...] = h2[:, 16:32]
    zs = jnp.zeros((B, 15), _f32)
    als2[...] = jnp.concatenate([h2 @ as2[...][0][:, None], zs], axis=1)
    ald2[...] = jnp.concatenate([h2 @ ad2[...][0][:, None], zs], axis=1)


def _tc3(S2ge, S2age, S2dc, S2adc, S2lat, dega, e_ge, e_age, e_dc, e_adc,
         g0, g1, g2, g3, den1, Wdge, Wddc, Wdlat, Wb, Wg2, as2, ad2):
    return pl.pallas_call(
        _tc3_body,
        grid=(GRID,),
        in_specs=[_row_spec(64)] * 5 + [_row_spec(16)] + [_row_spec(64)] * 8
                 + [_row_spec(16),
                    _full_spec(64, 128), _full_spec(64, 64), _full_spec(64, 128),
                    _full_spec(64, 64), _full_spec(256, 32),
                    _full_spec(1, 32), _full_spec(1, 32)],
        out_specs=[_row_spec(128), _row_spec(64), _row_spec(128), _row_spec(8),
                   _row_spec(16), _row_spec(16), _row_spec(16), _row_spec(16)],
        out_shape=[_sds(N, 128), _sds(N, 64), _sds(N, 128), _sds(N, 8),
                   _sds(N, 16), _sds(N, 16), _sds(N, 16), _sds(N, 16)],
    )(S2ge, S2age, S2dc, S2adc, S2lat, dega, e_ge, e_age, e_dc, e_adc,
      g0, g1, g2, g3, den1, Wdge, Wddc, Wdlat, Wb, Wg2, as2, ad2)


def _tc4_body(g2a, g2b, den2, x_out):
    den = den2[:, 0:1] + 1e-16
    x_out[...] = jnp.concatenate([g2a[...], g2b[...]], axis=1) / den


def _tc4(g2a, g2b, den2):
    return pl.pallas_call(
        _tc4_body,
        grid=(GRID,),
        in_specs=[_row_spec(16), _row_spec(16), _row_spec(16)],
        out_specs=_row_spec(32),
        out_shape=_sds(N, 32),
    )(g2a, g2b, den2)


_spmm_p1 = _make_spmm(2, 2, True)
_spmm_p2 = _make_spmm(3, 2, False)
_gat1 = _make_gat(64, 4, ((0, 0), (1, 1)), ((2, 2), (3, 3)))
_gat2 = _make_gat(16, 2, ((0, 0),), ((1, 0),))


def kernel(feat_ge, feat_a_ge, feat_deconv, feat_a_deconv, edge_index,
           W_enc_ge, W_dec_ge, W_enc_dc, W_dec_dc, W_b, W_gate, b_gate,
           W_dec_lat, W_g1, a_src1, a_dst1, W_g2, a_src2, a_dst2):
    src = edge_index[0]
    dst = edge_index[1]
    pad = EP - E
    srcp = jnp.concatenate([src, jnp.zeros((pad,), jnp.int32)]).reshape(ROWS_E, CHUNK)
    dstp = jnp.concatenate([dst, jnp.full((pad,), N, jnp.int32)]).reshape(ROWS_E, CHUNK)

    m_ge, m_age, m_dc, m_adc = _tc1(feat_ge, feat_a_ge, feat_deconv,
                                    feat_a_deconv, W_enc_ge, W_enc_dc)
    Sge, Sage, Sdc, Sadc, dega = _spmm_p1(m_ge, m_age, m_dc, m_adc, srcp, dstp)
    (e_ge, e_age, e_dc, e_adc, ge_n, dc_n, emb_lat, h0, h1, h2, h3,
     als, ald) = _tc2(Sge, Sage, Sdc, Sadc, dega, W_gate,
                      b_gate.reshape(1, 64), W_g1, a_src1, a_dst1)
    S2ge, S2age, S2dc, S2adc, S2lat = _spmm_p2(e_ge, e_age, e_dc, e_adc,
                                               emb_lat, srcp, dstp)
    g0, g1, g2, g3, den1 = _gat1(h0, h1, h2, h3, als, ald, srcp, dstp)
    (h_ge, h_dcv, h_lat, rets, h2a, h2b, als2, ald2) = _tc3(
        S2ge, S2age, S2dc, S2adc, S2lat, dega, e_ge, e_age, e_dc, e_adc,
        g0, g1, g2, g3, den1[0], W_dec_ge, W_dec_dc, W_dec_lat, W_b, W_g2,
        a_src2, a_dst2)
    g2a, g2b, den2 = _gat2(h2a, h2b, als2, ald2, srcp, dstp)
    x_out = _tc4(g2a, g2b, den2[0])

    return (ge_n, h_ge, rets[:, 0:2], rets[:, 2:4], dc_n, h_dcv,
            rets[:, 4:6], rets[:, 6:8], emb_lat, h_lat, x_out)


# speedup vs baseline: 24.9824x; 1.1976x over previous
import jax
import jax.numpy as jnp
from jax import lax
from jax.experimental import pallas as pl
from jax.experimental.pallas import tpu as pltpu
from jax.experimental.pallas import tpu_sc as plsc

N = 10000
E = 320000
CHUNK = 128
TILES = 16
CPT = 160
EP = CPT * CHUNK * TILES
ROWS_E = EP // CHUNK
N_ACC = TILES * 632
ZROWS = 632
OPT = 624
NH = TILES * 640
NHR = NH // 16
HZR = NHR // TILES
W = 64
B = 1000
GRID = N // B

_mesh = plsc.VectorSubcoreMesh(core_axis_name="c", subcore_axis_name="s")
_f32 = jnp.float32


def _sds(*shape):
    return jax.ShapeDtypeStruct(shape, _f32)


def _fill(ref, rows, width, value):
    v = jnp.full((16,), value, _f32)
    for r in range(rows):
        for k in range(width // 16):
            ref[r, pl.ds(k * 16, 16)] = v


def _zero_1d(ref, n):
    zv = jnp.zeros((16,), _f32)

    def stepz(i, cc):
        ref[pl.ds(i * 16, 16)] = zv
        return cc
    lax.fori_loop(0, n // 16, stepz, 0)


def _zero_rows(ref, nrows):
    zv = jnp.zeros((16,), _f32)

    def stepz(i, cc):
        ref[i] = zv
        return cc
    lax.fori_loop(0, nrows, stepz, 0)


def _fill_idx5(idx5):
    base = lax.iota(jnp.int32, 16)
    for p in range(5):
        for k in range(8):
            idx5[p, pl.ds(k * 16, 16)] = base + (p * 128 + k * 16)


def _zero_acc(zb, acc, s):
    nz = zb.shape[0]
    base = s * ZROWS
    for i in range((ZROWS + nz - 1) // nz):
        off = min(i * nz, ZROWS - nz)
        pltpu.sync_copy(zb, acc.at[pl.ds(base + off, nz)])


def _load_idx(srcp, dstp, src_v, dst_v, s):
    pltpu.sync_copy(srcp.at[pl.ds(s * CPT, CPT)], src_v)
    pltpu.sync_copy(dstp.at[pl.ds(s * CPT, CPT)], dst_v)


def _copy_rows(src, dst, s):
    row0 = s * OPT
    pltpu.sync_copy(src.at[pl.ds(row0, OPT)], dst.at[pl.ds(row0, OPT)])
    tail = N - TILES * OPT

    @pl.when(s == TILES - 1)
    def _():
        pltpu.sync_copy(src.at[pl.ds(TILES * OPT, tail)],
                        dst.at[pl.ds(TILES * OPT, tail)])


def _zero_hist(zrows, hist, s):
    pltpu.sync_copy(zrows, hist.at[pl.ds(s * HZR, HZR)])


def _reduce_hist(hist_l, hist, idx5, out2d, s):
    for p in range(5):
        pltpu.sync_copy(hist_l.at[pl.ds(p * 128, 128)],
                        hist.at[idx5.at[p]], add=True)
    plsc.subcore_barrier()
    row0 = s * HZR
    pltpu.sync_copy(hist.at[pl.ds(row0, HZR)], out2d.at[pl.ds(row0, HZR)])


def _make_spmm(n_x, n_out, groups_lo, groups_hi, with_deg):
    outs = [_sds(N, W) for _ in range(n_out)] + ([_sds(NHR, 16)] if with_deg else [])
    scratch = [
        pltpu.VMEM((CPT, CHUNK), jnp.int32),
        pltpu.VMEM((CPT, CHUNK), jnp.int32),
        pltpu.VMEM((CHUNK, W), _f32),
        pltpu.VMEM((CHUNK, W), _f32),
        pltpu.VMEM((8, W), _f32),
        pltpu.VMEM((NHR, 16), _f32),
        pltpu.VMEM((HZR, 16), _f32),
        pltpu.VMEM((5, 128), jnp.int32),
        pltpu.VMEM_SHARED((N_ACC, W), _f32),
        pltpu.VMEM_SHARED((NHR, 16), _f32),
        pltpu.SemaphoreType.DMA,
        pltpu.SemaphoreType.DMA,
    ]

    def body(*refs):
        xs = refs[:n_x]
        srcp, dstp = refs[n_x], refs[n_x + 1]
        o = n_x + 2
        outs_r = refs[o:o + n_out]
        o += n_out
        deg_out = refs[o] if with_deg else None
        o += 1 if with_deg else 0
        (src_v, dst_v, rows0, rows1, zb, deg_l, zrows, idx5, acc, hist,
         sem0, sem1) = refs[o:]
        c = lax.axis_index("c")
        s = lax.axis_index("s")
        _load_idx(srcp, dstp, src_v, dst_v, s)
        _fill(zb, 8, W, 0.0)
        if with_deg:
            _zero_rows(zrows, HZR)
            _fill_idx5(idx5)
        ones16 = jnp.ones((16,), _f32)

        def run_group(x_ref, out_ref, c0, c1, do_deg):
            _zero_acc(zb, acc, s)
            if do_deg:
                _zero_rows(deg_l, NHR)
                _zero_hist(zrows, hist, s)
            plsc.subcore_barrier()

            def issue(j, buf, sem):
                pltpu.async_copy(x_ref.at[src_v.at[j]], buf, sem)

            def drain(j, buf, sem):
                pltpu.make_async_copy(x_ref.at[src_v.at[0]], buf, sem).wait()
                if do_deg:
                    def stepd(q, cc):
                        didx = dst_v[j, pl.ds(q * 16, 16)]
                        plsc.addupdate_scatter(
                            deg_l, [jnp.right_shift(didx, 4),
                                    jnp.bitwise_and(didx, 15)], ones16)
                        return cc
                    lax.fori_loop(0, CHUNK // 16, stepd, 0)
                pltpu.sync_copy(buf, acc.at[dst_v.at[j]], add=True)

            issue(c0, rows0, sem0)
            issue(c0 + 1, rows1, sem1)

            def step(jj, carry):
                j0 = c0 + 2 * jj
                drain(j0, rows0, sem0)

                @pl.when(j0 + 2 < c1)
                def _():
                    issue(j0 + 2, rows0, sem0)

                drain(j0 + 1, rows1, sem1)

                @pl.when(j0 + 3 < c1)
                def _():
                    issue(j0 + 3, rows1, sem1)
                return carry
            lax.fori_loop(0, (c1 - c0) // 2, step, 0)
            plsc.subcore_barrier()
            _copy_rows(acc, out_ref, s)
            if do_deg:
                _reduce_hist(deg_l, hist, idx5, deg_out, s)

        @pl.when(c == 0)
        def _():
            for (xi, oi, c0, c1, dd) in groups_lo:
                run_group(xs[xi], outs_r[oi], c0, c1, dd and with_deg)

        @pl.when(c == 1)
        def _():
            for (xi, oi, c0, c1, dd) in groups_hi:
                run_group(xs[xi], outs_r[oi], c0, c1, False)

    return pl.kernel(body, out_type=tuple(outs), mesh=_mesh, scratch_types=scratch,
                     compiler_params=pltpu.CompilerParams(use_tc_tiling_on_sc=False, needs_layout_passes=False))


def _make_gat(hw, n_heads, heads_lo, heads_hi, den_hi):
    outs = [_sds(N, hw) for _ in range(n_heads)] + [_sds(n_heads, NHR, 16)]
    scratch = [
        pltpu.VMEM((CPT, CHUNK), jnp.int32),
        pltpu.VMEM((CPT, CHUNK), jnp.int32),
        pltpu.VMEM((10016,), _f32),
        pltpu.VMEM((10016,), _f32),
        pltpu.VMEM((NHR, 16), _f32),
        pltpu.VMEM((HZR, 16), _f32),
        pltpu.VMEM((5, 128), jnp.int32),
        pltpu.VMEM((CHUNK, hw), _f32),
        pltpu.VMEM((CHUNK, hw), _f32),
        pltpu.VMEM((8, hw), _f32),
        pltpu.VMEM_SHARED((N_ACC, hw), _f32),
        pltpu.VMEM_SHARED((NHR, 16), _f32),
        pltpu.SemaphoreType.DMA,
        pltpu.SemaphoreType.DMA,
    ]

    def body(*refs):
        xs = refs[:n_heads]
        alsT, aldT, srcp, dstp = refs[n_heads:n_heads + 4]
        o = n_heads + 4
        outs_r = refs[o:o + n_heads]
        den_out = refs[o + n_heads]
        (src_v, dst_v, als_l, ald_l, den_l, zrows, idx5, h0, h1, zb, acc,
         hist, sem0, sem1) = refs[o + n_heads + 1:]
        c = lax.axis_index("c")
        s = lax.axis_index("s")
        _load_idx(srcp, dstp, src_v, dst_v, s)
        _fill(zb, 8, hw, 0.0)
        _zero_rows(zrows, HZR)
        _fill_idx5(idx5)

        def run_head(head, do_den):
            x_ref = xs[head]
            _zero_acc(zb, acc, s)
            if do_den:
                _zero_rows(den_l, NHR)
                _zero_hist(zrows, hist, s)
            pltpu.sync_copy(alsT.at[head], als_l.at[pl.ds(0, N)])
            pltpu.sync_copy(aldT.at[head], ald_l.at[pl.ds(0, N)])
            plsc.subcore_barrier()

            def issue(j, buf, sem):
                pltpu.async_copy(x_ref.at[src_v.at[j]], buf, sem)

            def drain(j, buf, sem):
                pltpu.make_async_copy(x_ref.at[src_v.at[0]], buf, sem).wait()

                def stepq(q, cc):
                    sidx = src_v[j, pl.ds(q * 16, 16)]
                    didx = dst_v[j, pl.ds(q * 16, 16)]
                    av = plsc.load_gather(als_l, [sidx])
                    bv = plsc.load_gather(ald_l, [didx])
                    a = av + bv
                    a = jnp.where(a > 0.0, a, 0.2 * a)
                    s16 = jnp.exp(a)
                    if do_den:
                        plsc.addupdate_scatter(
                            den_l, [jnp.right_shift(didx, 4),
                                    jnp.bitwise_and(didx, 15)], s16)
                    for u in range(16):
                        e = q * 16 + u
                        se = s16[u]
                        for k in range(hw // 16):
                            buf[e, pl.ds(k * 16, 16)] = (
                                buf[e, pl.ds(k * 16, 16)] * se)
                    return cc
                lax.fori_loop(0, CHUNK // 16, stepq, 0)
                pltpu.sync_copy(buf, acc.at[dst_v.at[j]], add=True)

            issue(0, h0, sem0)
            issue(1, h1, sem1)

            def step(jj, carry):
                j0 = 2 * jj
                drain(j0, h0, sem0)

                @pl.when(j0 + 2 < CPT)
                def _():
                    issue(j0 + 2, h0, sem0)

                drain(j0 + 1, h1, sem1)

                @pl.when(j0 + 3 < CPT)
                def _():
                    issue(j0 + 3, h1, sem1)
                return carry
            lax.fori_loop(0, CPT // 2, step, 0)
            plsc.subcore_barrier()
            _copy_rows(acc, outs_r[head], s)
            if do_den:
                _reduce_hist(den_l, hist, idx5, den_out.at[head], s)

        @pl.when(c == 0)
        def _():
            for head in heads_lo:
                run_head(head, True)

        @pl.when(c == 1)
        def _():
            for head in heads_hi:
                run_head(head, den_hi)

    return pl.kernel(body, out_type=tuple(outs), mesh=_mesh, scratch_types=scratch,
                     compiler_params=pltpu.CompilerParams(use_tc_tiling_on_sc=False, needs_layout_passes=False))


def _row_spec(w):
    return pl.BlockSpec((B, w), lambda i: (i, 0))


def _full_spec(a, b):
    return pl.BlockSpec((a, b), lambda i: (0, 0))


def _tc1_body(fg, fa, fd, fad, Wge, Wdc, m0, m1, m2, m3):
    m0[...] = fg[...] @ Wge[...]
    m1[...] = fa[...] @ Wge[...]
    m2[...] = fd[...] @ Wdc[...]
    m3[...] = fad[...] @ Wdc[...]


def _tc1(fg, fa, fd, fad, Wge, Wdc):
    return pl.pallas_call(
        _tc1_body,
        grid=(GRID,),
        in_specs=[_row_spec(128), _row_spec(128), _row_spec(64), _row_spec(64),
                  _full_spec(128, 64), _full_spec(64, 64)],
        out_specs=[_row_spec(64)] * 4,
        out_shape=[_sds(N, 64)] * 4,
    )(fg, fa, fd, fad, Wge, Wdc)


def _l2n(x):
    return x / jnp.clip(jnp.sqrt(jnp.sum(x * x, axis=-1, keepdims=True)), 1e-12)


def _tc2_body(Sge, Sage, Sdc, Sadc, dega, Wg, bg, Wg1, as1, ad1,
              e_ge_o, e_age_o, e_dc_o, e_adc_o, ge_n_o, dc_n_o, lat_o,
              h0, h1, h2, h3, als_o, ald_o):
    deg = jnp.maximum(dega[...], 1.0)
    e_ge = jax.nn.relu(Sge[...] / deg)
    e_age = jax.nn.relu(Sage[...] / deg)
    e_dc = jax.nn.relu(Sdc[...] / deg)
    e_adc = jax.nn.relu(Sadc[...] / deg)
    e_ge_o[...] = e_ge
    e_age_o[...] = e_age
    e_dc_o[...] = e_dc
    e_adc_o[...] = e_adc
    gn = _l2n(e_ge)
    dn = _l2n(e_dc)
    ge_n_o[...] = gn
    dc_n_o[...] = dn
    gate = jax.nn.sigmoid(jnp.concatenate([gn, dn], axis=1) @ Wg[...] + bg[...])
    el = gate * gn + (1.0 - gate) * dn
    lat_o[...] = el
    h = el @ Wg1[...]
    h0[...] = h[:, 0:64]
    h1[...] = h[:, 64:128]
    h2[...] = h[:, 128:192]
    h3[...] = h[:, 192:256]
    zs = jnp.zeros((B, 12), _f32)
    als_o[...] = jnp.concatenate(
        [h[:, 64 * k:64 * k + 64] @ as1[...][k][:, None] for k in range(4)] + [zs],
        axis=1)
    ald_o[...] = jnp.concatenate(
        [h[:, 64 * k:64 * k + 64] @ ad1[...][k][:, None] for k in range(4)] + [zs],
        axis=1)


def _tc2(Sge, Sage, Sdc, Sadc, dega, Wg, bg, Wg1, as1, ad1):
    return pl.pallas_call(
        _tc2_body,
        grid=(GRID,),
        in_specs=[_row_spec(64)] * 4 + [_row_spec(1),
                  _full_spec(128, 64), _full_spec(1, 64), _full_spec(64, 256),
                  _full_spec(4, 64), _full_spec(4, 64)],
        out_specs=[_row_spec(64)] * 11 + [_row_spec(16), _row_spec(16)],
        out_shape=[_sds(N, 64)] * 11 + [_sds(N, 16), _sds(N, 16)],
    )(Sge, Sage, Sdc, Sadc, dega, Wg, bg, Wg1, as1, ad1)


def _tc3_body(S2ge, S2age, S2dc, S2adc, S2lata, S2latb, dega,
              e_ge, e_age, e_dc, e_adc,
              g0, g1, g2, g3, den1, Wdge, Wddc, Wdlat, Wb, Wg2, as2, ad2,
              h_ge_o, h_dcv_o, h_lat_o, rets, h2a, h2b, als2, ald2):
    deg = jnp.maximum(dega[...], 1.0)
    Pge = S2ge[...] / deg
    Page = S2age[...] / deg
    Pdc = S2dc[...] / deg
    Padc = S2adc[...] / deg
    Plat = (S2lata[...] + S2latb[...]) / deg
    h_ge_o[...] = Pge @ Wdge[...]
    h_dcv_o[...] = Pdc @ Wddc[...]
    h_lat_o[...] = Plat @ Wdlat[...]

    def disc(g, hp, hm):
        cw = g @ Wb[...]
        return (jnp.sum(hp * cw, axis=-1, keepdims=True),
                jnp.sum(hm * cw, axis=-1, keepdims=True))

    a1, a2 = disc(jax.nn.sigmoid(Pge), e_ge[...], e_age[...])
    b1, b2 = disc(jax.nn.sigmoid(Page), e_age[...], e_ge[...])
    c1, c2 = disc(jax.nn.sigmoid(Pdc), e_dc[...], e_adc[...])
    d1, d2 = disc(jax.nn.sigmoid(Padc), e_adc[...], e_dc[...])
    rets[...] = jnp.concatenate([a1, a2, b1, b2, c1, c2, d1, d2], axis=1)
    den = den1[...] + 1e-16
    h1v = jnp.concatenate(
        [g[...] / den[:, k:k + 1] for k, g in enumerate((g0, g1, g2, g3))], axis=1)
    h1v = jnp.where(h1v > 0.0, h1v, jnp.exp(jnp.minimum(h1v, 0.0)) - 1.0)
    h2 = h1v @ Wg2[...]
    h2a[...] = h2[:, 0:16]
    h2b[...] = h2[:, 16:32]
    als2[...] = h2 @ as2[...][0][:, None]
    ald2[...] = h2 @ ad2[...][0][:, None]


def _tc3(S2ge, S2age, S2dc, S2adc, S2lata, S2latb, dega, e_ge, e_age, e_dc,
         e_adc, g0, g1, g2, g3, den1, Wdge, Wddc, Wdlat, Wb, Wg2, as2, ad2):
    return pl.pallas_call(
        _tc3_body,
        grid=(GRID,),
        in_specs=[_row_spec(64)] * 6 + [_row_spec(1)] + [_row_spec(64)] * 8
                 + [_row_spec(4),
                    _full_spec(64, 128), _full_spec(64, 64), _full_spec(64, 128),
                    _full_spec(64, 64), _full_spec(256, 32),
                    _full_spec(1, 32), _full_spec(1, 32)],
        out_specs=[_row_spec(128), _row_spec(64), _row_spec(128), _row_spec(8),
                   _row_spec(16), _row_spec(16), _row_spec(1), _row_spec(1)],
        out_shape=[_sds(N, 128), _sds(N, 64), _sds(N, 128), _sds(N, 8),
                   _sds(N, 16), _sds(N, 16), _sds(N, 1), _sds(N, 1)],
    )(S2ge, S2age, S2dc, S2adc, S2lata, S2latb, dega, e_ge, e_age, e_dc, e_adc,
      g0, g1, g2, g3, den1, Wdge, Wddc, Wdlat, Wb, Wg2, as2, ad2)


def _tc4_body(g2a, g2b, den2, x_out):
    den = den2[...] + 1e-16
    x_out[...] = jnp.concatenate([g2a[...], g2b[...]], axis=1) / den


def _tc4(g2a, g2b, den2):
    return pl.pallas_call(
        _tc4_body,
        grid=(GRID,),
        in_specs=[_row_spec(16), _row_spec(16), _row_spec(1)],
        out_specs=_row_spec(32),
        out_shape=_sds(N, 32),
    )(g2a, g2b, den2)


_spmm_p1 = _make_spmm(
    4, 4,
    groups_lo=((0, 0, 0, CPT, True), (1, 1, 0, CPT, False)),
    groups_hi=((2, 2, 0, CPT, False), (3, 3, 0, CPT, False)),
    with_deg=True)
_spmm_p2 = _make_spmm(
    5, 6,
    groups_lo=((0, 0, 0, CPT, False), (1, 1, 0, CPT, False),
               (4, 4, 0, CPT // 2, False)),
    groups_hi=((2, 2, 0, CPT, False), (3, 3, 0, CPT, False),
               (4, 5, CPT // 2, CPT, False)),
    with_deg=False)
_gat1 = _make_gat(64, 4, (0, 1), (2, 3), True)
_gat2 = _make_gat(16, 2, (0,), (1,), False)


def kernel(feat_ge, feat_a_ge, feat_deconv, feat_a_deconv, edge_index,
           W_enc_ge, W_dec_ge, W_enc_dc, W_dec_dc, W_b, W_gate, b_gate,
           W_dec_lat, W_g1, a_src1, a_dst1, W_g2, a_src2, a_dst2):
    src = edge_index[0]
    dst = edge_index[1]
    pad = EP - E
    srcp = jnp.concatenate([src, jnp.zeros((pad,), jnp.int32)]).reshape(ROWS_E, CHUNK)
    dstp = jnp.concatenate([dst, jnp.full((pad,), N, jnp.int32)]).reshape(ROWS_E, CHUNK)

    m_ge, m_age, m_dc, m_adc = _tc1(feat_ge, feat_a_ge, feat_deconv,
                                    feat_a_deconv, W_enc_ge, W_enc_dc)
    Sge, Sage, Sdc, Sadc, deg2d = _spmm_p1(m_ge, m_age, m_dc, m_adc, srcp, dstp)
    dega = deg2d.reshape(NH)[:N].reshape(N, 1)
    (e_ge, e_age, e_dc, e_adc, ge_n, dc_n, emb_lat, h0, h1, h2, h3,
     als, ald) = _tc2(Sge, Sage, Sdc, Sadc, dega, W_gate,
                      b_gate.reshape(1, 64), W_g1, a_src1, a_dst1)
    S2ge, S2age, S2dc, S2adc, S2lata, S2latb = _spmm_p2(
        e_ge, e_age, e_dc, e_adc, emb_lat, srcp, dstp)
    alsT = als.T[0:4]
    aldT = ald.T[0:4]
    g0, g1, g2, g3, den1T = _gat1(h0, h1, h2, h3, alsT, aldT, srcp, dstp)
    den1 = den1T.reshape(4, NH)[:, :N].T
    (h_ge, h_dcv, h_lat, rets, h2a, h2b, als2, ald2) = _tc3(
        S2ge, S2age, S2dc, S2adc, S2lata, S2latb, dega, e_ge, e_age, e_dc,
        e_adc, g0, g1, g2, g3, den1, W_dec_ge, W_dec_dc, W_dec_lat, W_b,
        W_g2, a_src2, a_dst2)
    g2a, g2b, den2T = _gat2(h2a, h2b, als2.reshape(1, N), ald2.reshape(1, N),
                            srcp, dstp)
    x_out = _tc4(g2a, g2b, den2T[0].reshape(NH)[:N].reshape(N, 1))

    return (ge_n, h_ge, rets[:, 0:2], rets[:, 2:4], dc_n, h_dcv,
            rets[:, 4:6], rets[:, 6:8], emb_lat, h_lat, x_out)
